# baseline (speedup 1.0000x reference)
import jax
import jax.numpy as jnp
from jax import lax
from jax.experimental import pallas as pl
from jax.experimental.pallas import tpu as pltpu

MASK_LEN = 4915
BT = 512


def _body(start_ref, x_ref, mt_ref, out_ref):
    b = pl.program_id(0)
    tb = pl.program_id(1)
    s = start_ref[b]
    base = tb * BT
    pos_col = lax.broadcasted_iota(jnp.int32, (BT, 1), 0) + base
    m_col = (pos_col >= s) & (pos_col < s + MASK_LEN)
    out_ref[0] = jnp.where(m_col, mt_ref[0][None, :], x_ref[0])


def _mask_body(start_ref, mask_ref):
    B, T = mask_ref.shape
    pos = lax.broadcasted_iota(jnp.int32, (1, T), 1)
    for b in range(B):
        s = start_ref[b]
        mask_ref[b : b + 1, :] = (pos >= s) & (pos < s + MASK_LEN)


def kernel(x, start_idx, mask_token):
    B, T, D = x.shape
    grid_spec = pltpu.PrefetchScalarGridSpec(
        num_scalar_prefetch=1,
        grid=(B, T // BT),
        in_specs=[
            pl.BlockSpec((1, BT, D), lambda b, t, s: (b, t, 0)),
            pl.BlockSpec((1, D), lambda b, t, s: (0, 0)),
        ],
        out_specs=[
            pl.BlockSpec((1, BT, D), lambda b, t, s: (b, t, 0)),
        ],
    )
    x_masked = pl.pallas_call(
        _body,
        grid_spec=grid_spec,
        out_shape=[jax.ShapeDtypeStruct((B, T, D), x.dtype)],
    )(start_idx, x, mask_token.reshape(1, D))[0]
    mask = pl.pallas_call(
        _mask_body,
        in_specs=[pl.BlockSpec(memory_space=pltpu.SMEM)],
        out_shape=jax.ShapeDtypeStruct((B, T), jnp.bool_),
    )(start_idx)
    return (x_masked, mask)
